# baseline (speedup 1.0000x reference)
import jax
import jax.numpy as jnp
from jax.experimental import pallas as pl

_L = 25600

_CC = (((0,), (0,)), ((), ()))
_MM = (((1,), (0,)), ((), ()))


def _sig(z):
    return 0.5 * jnp.tanh(0.5 * z) + 0.5


def _cell_kernel(x_ref, hc_ref, w_ref, out_ref, hout_ref, cout_ref):
    xb = x_ref[...]
    hc = hc_ref[...]
    w = w_ref[...]
    cb = hc[3:6, :]

    def zgate(g):
        zx = jax.lax.dot_general(w[:, 3 * g:3 * g + 3], xb, _CC,
                                 preferred_element_type=jnp.float32)
        zh = jax.lax.dot_general(w[0:7, 12 + 3 * g:15 + 3 * g], hc, _CC,
                                 preferred_element_type=jnp.float32)
        return zx + zh

    gi = _sig(zgate(0))
    gf = _sig(zgate(1))
    gt = jnp.tanh(zgate(2))
    c_new = gf * cb + gi * gt
    zo = zgate(3) + jax.lax.dot_general(w[0:3, 24:27], c_new, _CC,
                                        preferred_element_type=jnp.float32)
    go = _sig(zo)
    h_new = go * jnp.tanh(c_new)
    orow = (jax.lax.dot_general(
        w[4:5, 24:27], jax.nn.relu(h_new), _MM,
        preferred_element_type=jnp.float32) + w[5:6, 24:25])
    out_ref[...] = orow[0, :]
    hout_ref[...] = h_new
    cout_ref[...] = c_new


def kernel(x, edge_index, edge_weight, h, c,
           W_i, W_f, W_c, W_o,
           Th_i, Th_f, Th_c, Th_o,
           cb_i, cb_f, cb_c, cb_o,
           b_i, b_f, b_c, b_o,
           wc_i, wc_f, wc_o,
           lin_W, lin_b):
    n = x.shape[0]
    f32 = jnp.float32
    xt = x.T
    hcb = jnp.concatenate(
        [h, c, jnp.ones((n, 1), f32)], axis=1).T

    def put(a, r0, c0):
        return jnp.pad(a, ((r0, 12 - r0 - a.shape[0]),
                           (c0, 28 - c0 - a.shape[1])))

    eye3 = jnp.eye(3, dtype=f32)
    gates = ((W_i, Th_i, cb_i, b_i), (W_f, Th_f, cb_f, b_f),
             (W_c, Th_c, cb_c, b_c), (W_o, Th_o, cb_o, b_o))
    pieces = []
    for g, (wg, tg, cbg, bg) in enumerate(gates):
        pieces.append(put(wg, 0, 3 * g))
        pieces.append(put(tg, 0, 12 + 3 * g))
        pieces.append(put(cbg[None, :] + bg, 6, 12 + 3 * g))
    pieces.append(put(wc_i * eye3, 3, 12))
    pieces.append(put(wc_f * eye3, 3, 15))
    pieces.append(put(wc_o * eye3, 0, 24))
    pieces.append(put(lin_W, 4, 24))
    pieces.append(put(lin_b.reshape(1, 1), 5, 24))
    w_all = sum(pieces)

    grid = (pl.cdiv(n, _L),)
    lane = lambda r: pl.BlockSpec((r, _L), lambda i: (0, i))

    outv, ht, ct = pl.pallas_call(
        _cell_kernel,
        grid=grid,
        in_specs=[lane(12), lane(7),
                  pl.BlockSpec((12, 28), lambda i: (0, 0))],
        out_specs=[pl.BlockSpec((_L,), lambda i: (i,)), lane(3), lane(3)],
        out_shape=[
            jax.ShapeDtypeStruct((n,), f32),
            jax.ShapeDtypeStruct((3, n), f32),
            jax.ShapeDtypeStruct((3, n), f32),
        ],
    )(xt, hcb, w_all)
    return (outv.reshape(n, 1), ht.T, ct.T)


# speedup vs baseline: 1.0863x; 1.0863x over previous
import jax
import jax.numpy as jnp
from jax.experimental import pallas as pl

_L = 25600

_CC = (((0,), (0,)), ((), ()))
_MM = (((1,), (0,)), ((), ()))


def _sig(z):
    return 0.5 * jnp.tanh(0.5 * z) + 0.5


def _cell_kernel(x_ref, hc_ref, w_ref, out_ref, hout_ref, cout_ref):
    xb = x_ref[...]
    hc = hc_ref[...]
    w = w_ref[...]
    cb = hc[3:6, :]

    def zgate(g):
        zx = jax.lax.dot_general(w[:, 3 * g:3 * g + 3], xb, _CC,
                                 preferred_element_type=jnp.float32)
        zh = jax.lax.dot_general(w[0:7, 12 + 3 * g:15 + 3 * g], hc, _CC,
                                 preferred_element_type=jnp.float32)
        return zx + zh

    gi = _sig(zgate(0))
    gf = _sig(zgate(1))
    gt = jnp.tanh(zgate(2))
    c_new = gf * cb + gi * gt
    zo = zgate(3) + jax.lax.dot_general(w[0:3, 24:27], c_new, _CC,
                                        preferred_element_type=jnp.float32)
    go = _sig(zo)
    h_new = go * jnp.tanh(c_new)
    orow = (jax.lax.dot_general(
        w[4:5, 24:27], jax.nn.relu(h_new), _MM,
        preferred_element_type=jnp.float32) + w[5:6, 24:25])
    out_ref[...] = orow
    hout_ref[...] = h_new
    cout_ref[...] = c_new


def kernel(x, edge_index, edge_weight, h, c,
           W_i, W_f, W_c, W_o,
           Th_i, Th_f, Th_c, Th_o,
           cb_i, cb_f, cb_c, cb_o,
           b_i, b_f, b_c, b_o,
           wc_i, wc_f, wc_o,
           lin_W, lin_b):
    n = x.shape[0]
    f32 = jnp.float32
    xt = x.T
    hcb = jnp.concatenate(
        [h, c, jnp.ones((n, 1), f32)], axis=1).T

    def put(a, r0, c0):
        return jnp.pad(a, ((r0, 12 - r0 - a.shape[0]),
                           (c0, 28 - c0 - a.shape[1])))

    eye3 = jnp.eye(3, dtype=f32)
    gates = ((W_i, Th_i, cb_i, b_i), (W_f, Th_f, cb_f, b_f),
             (W_c, Th_c, cb_c, b_c), (W_o, Th_o, cb_o, b_o))
    pieces = []
    for g, (wg, tg, cbg, bg) in enumerate(gates):
        pieces.append(put(wg, 0, 3 * g))
        pieces.append(put(tg, 0, 12 + 3 * g))
        pieces.append(put(cbg[None, :] + bg, 6, 12 + 3 * g))
    pieces.append(put(wc_i * eye3, 3, 12))
    pieces.append(put(wc_f * eye3, 3, 15))
    pieces.append(put(wc_o * eye3, 0, 24))
    pieces.append(put(lin_W, 4, 24))
    pieces.append(put(lin_b.reshape(1, 1), 5, 24))
    w_all = sum(pieces)

    grid = (pl.cdiv(n, _L),)
    lane = lambda r: pl.BlockSpec((r, _L), lambda i: (0, i))

    outv, ht, ct = pl.pallas_call(
        _cell_kernel,
        grid=grid,
        in_specs=[lane(12), lane(7),
                  pl.BlockSpec((12, 28), lambda i: (0, 0))],
        out_specs=[lane(1), lane(3), lane(3)],
        out_shape=[
            jax.ShapeDtypeStruct((1, n), f32),
            jax.ShapeDtypeStruct((3, n), f32),
            jax.ShapeDtypeStruct((3, n), f32),
        ],
    )(xt, hcb, w_all)
    return (outv.T, ht.T, ct.T)


# speedup vs baseline: 1.1207x; 1.0317x over previous
import jax
import jax.numpy as jnp
from jax.experimental import pallas as pl

_L = 25600

_CC = (((0,), (0,)), ((), ()))
_MM = (((1,), (0,)), ((), ()))


def _sig(z):
    return 0.5 * jnp.tanh(0.5 * z) + 0.5


def _cell_kernel(x_ref, hc_ref, w_ref, out_ref, hout_ref, cout_ref):
    xb = x_ref[...]
    hc = hc_ref[...]
    w = w_ref[...]
    cb = hc[3:6, :]

    z = (jax.lax.dot_general(w[:, 0:12], xb, _CC,
                             preferred_element_type=jnp.float32)
         + jax.lax.dot_general(w[0:7, 12:24], hc, _CC,
                               preferred_element_type=jnp.float32))
    gi = _sig(z[0:3, :])
    gf = _sig(z[3:6, :])
    gt = jnp.tanh(z[6:9, :])
    c_new = gf * cb + gi * gt
    zo = z[9:12, :] + jax.lax.dot_general(w[0:3, 24:27], c_new, _CC,
                                          preferred_element_type=jnp.float32)
    go = _sig(zo)
    h_new = go * jnp.tanh(c_new)
    orow = (jax.lax.dot_general(
        w[4:5, 24:27], jax.nn.relu(h_new), _MM,
        preferred_element_type=jnp.float32) + w[5:6, 24:25])
    out_ref[...] = orow
    hout_ref[...] = h_new
    cout_ref[...] = c_new


def kernel(x, edge_index, edge_weight, h, c,
           W_i, W_f, W_c, W_o,
           Th_i, Th_f, Th_c, Th_o,
           cb_i, cb_f, cb_c, cb_o,
           b_i, b_f, b_c, b_o,
           wc_i, wc_f, wc_o,
           lin_W, lin_b):
    n = x.shape[0]
    f32 = jnp.float32
    xt = x.T
    hcb = jnp.concatenate(
        [h, c, jnp.ones((n, 1), f32)], axis=1).T

    def put(a, r0, c0):
        return jnp.pad(a, ((r0, 12 - r0 - a.shape[0]),
                           (c0, 28 - c0 - a.shape[1])))

    eye3 = jnp.eye(3, dtype=f32)
    gates = ((W_i, Th_i, cb_i, b_i), (W_f, Th_f, cb_f, b_f),
             (W_c, Th_c, cb_c, b_c), (W_o, Th_o, cb_o, b_o))
    pieces = []
    for g, (wg, tg, cbg, bg) in enumerate(gates):
        pieces.append(put(wg, 0, 3 * g))
        pieces.append(put(tg, 0, 12 + 3 * g))
        pieces.append(put(cbg[None, :] + bg, 6, 12 + 3 * g))
    pieces.append(put(wc_i * eye3, 3, 12))
    pieces.append(put(wc_f * eye3, 3, 15))
    pieces.append(put(wc_o * eye3, 0, 24))
    pieces.append(put(lin_W, 4, 24))
    pieces.append(put(lin_b.reshape(1, 1), 5, 24))
    w_all = sum(pieces)

    grid = (pl.cdiv(n, _L),)
    lane = lambda r: pl.BlockSpec((r, _L), lambda i: (0, i))

    outv, ht, ct = pl.pallas_call(
        _cell_kernel,
        grid=grid,
        in_specs=[lane(12), lane(7),
                  pl.BlockSpec((12, 28), lambda i: (0, 0))],
        out_specs=[lane(1), lane(3), lane(3)],
        out_shape=[
            jax.ShapeDtypeStruct((1, n), f32),
            jax.ShapeDtypeStruct((3, n), f32),
            jax.ShapeDtypeStruct((3, n), f32),
        ],
    )(xt, hcb, w_all)
    return (outv.T, ht.T, ct.T)


# speedup vs baseline: 1.1953x; 1.0665x over previous
import jax
import jax.numpy as jnp
from jax.experimental import pallas as pl

_L = 25600

_CC = (((0,), (0,)), ((), ()))
_MM = (((1,), (0,)), ((), ()))


def _sig(z):
    return 0.5 * jnp.tanh(0.5 * z) + 0.5


def _cell_kernel(x_ref, hc_ref, w_ref, out_ref, hout_ref, cout_ref):
    xb = x_ref[...]
    hc = hc_ref[...]
    w = w_ref[...]
    cb = hc[3:6, :]

    z = (jax.lax.dot_general(w[:, 0:12], xb, _CC,
                             preferred_element_type=jnp.float32)
         + jax.lax.dot_general(w[0:7, 12:24], hc, _CC,
                               preferred_element_type=jnp.float32))
    gi = _sig(z[0:3, :])
    gf = _sig(z[3:6, :])
    gt = jnp.tanh(z[6:9, :])
    c_new = gf * cb + gi * gt
    zo = z[9:12, :] + jax.lax.dot_general(w[0:3, 24:27], c_new, _CC,
                                          preferred_element_type=jnp.float32)
    go = _sig(zo)
    h_new = go * jnp.tanh(c_new)
    orow = (jax.lax.dot_general(
        w[4:5, 24:27], jax.nn.relu(h_new), _MM,
        preferred_element_type=jnp.float32) + w[5:6, 24:25])
    out_ref[...] = orow
    hout_ref[...] = h_new
    cout_ref[...] = c_new


def kernel(x, edge_index, edge_weight, h, c,
           W_i, W_f, W_c, W_o,
           Th_i, Th_f, Th_c, Th_o,
           cb_i, cb_f, cb_c, cb_o,
           b_i, b_f, b_c, b_o,
           wc_i, wc_f, wc_o,
           lin_W, lin_b):
    n = x.shape[0]
    f32 = jnp.float32
    xt = x.T
    hcb = jnp.concatenate(
        [h, c, jnp.ones((n, 1), f32)], axis=1).T

    def put(a, r0, c0):
        return jnp.pad(a, ((r0, 12 - r0 - a.shape[0]),
                           (c0, 28 - c0 - a.shape[1])))

    eye3 = jnp.eye(3, dtype=f32)
    gates = ((W_i, Th_i, cb_i, b_i), (W_f, Th_f, cb_f, b_f),
             (W_c, Th_c, cb_c, b_c), (W_o, Th_o, cb_o, b_o))
    pieces = []
    for g, (wg, tg, cbg, bg) in enumerate(gates):
        pieces.append(put(wg, 0, 3 * g))
        pieces.append(put(tg, 0, 12 + 3 * g))
        pieces.append(put(cbg[None, :], 6, 12 + 3 * g))
        pieces.append(put(bg, 6, 12 + 3 * g))
    pieces.append(put(wc_i * eye3, 3, 12))
    pieces.append(put(wc_f * eye3, 3, 15))
    pieces.append(put(wc_o * eye3, 0, 24))
    pieces.append(put(lin_W, 4, 24))
    pieces.append(put(lin_b.reshape(1, 1), 5, 24))
    w_all = sum(pieces)

    grid = (pl.cdiv(n, _L),)
    lane = lambda r: pl.BlockSpec((r, _L), lambda i: (0, i))

    outv, ht, ct = pl.pallas_call(
        _cell_kernel,
        grid=grid,
        in_specs=[lane(12), lane(7),
                  pl.BlockSpec((12, 28), lambda i: (0, 0))],
        out_specs=[lane(1), lane(3), lane(3)],
        out_shape=[
            jax.ShapeDtypeStruct((1, n), f32),
            jax.ShapeDtypeStruct((3, n), f32),
            jax.ShapeDtypeStruct((3, n), f32),
        ],
    )(xt, hcb, w_all)
    return (outv.T, ht.T, ct.T)
